# baseline (speedup 1.0000x reference)
import functools
import math

import jax
import jax.numpy as jnp
from jax import lax
from jax.experimental import pallas as pl
from jax.experimental.pallas import tpu as pltpu
from jax.experimental.pallas import tpu_sc as plsc

_L = 16
_NC = 2
_NS = 16


@functools.partial(jax.jit, static_argnames=("h", "w"))
def _pos_embed_sc(row_embed, col_embed, h, w):
    d_model = row_embed.shape[1]
    ib_n, jb_n = 8, 4
    ipw = h // ib_n
    jpw = w // jb_n
    nq = d_model // _L

    mesh = plsc.VectorSubcoreMesh(
        core_axis_name="c", subcore_axis_name="s",
        num_cores=_NC, num_subcores=_NS,
    )

    @functools.partial(
        pl.kernel,
        out_type=jax.ShapeDtypeStruct((h, w, d_model), jnp.float32),
        mesh=mesh,
        scratch_types=[
            pltpu.VMEM((ipw, d_model), jnp.float32),
            pltpu.VMEM((jpw, d_model), jnp.float32),
            pltpu.VMEM((ipw, jpw, d_model), jnp.float32),
            pltpu.SemaphoreType.DMA,
            pltpu.SemaphoreType.DMA,
        ],
        compiler_params=pltpu.CompilerParams(
            needs_layout_passes=False,
            disable_bounds_checks=True,
            disable_semaphore_checks=True,
        ),
    )
    def body(col_hbm, row_hbm, out_hbm, colv, rowv, outv, sem_in, sem_out):
        wid = lax.axis_index("s") * _NC + lax.axis_index("c")
        ib = wid // jb_n
        jb = wid - ib * jb_n
        i0 = ib * ipw
        j0 = jb * jpw
        cin0 = pltpu.async_copy(col_hbm.at[pl.ds(i0, ipw)], colv, sem_in)
        cin1 = pltpu.async_copy(row_hbm.at[pl.ds(j0, jpw)], rowv, sem_in)
        cin0.wait()
        cin1.wait()

        half = ipw // 2
        outs = []
        for hi in range(2):
            def ig_body(ig, _, hi=hi):
                i = hi * half + ig * 2
                av0 = [colv[i, pl.ds(q * _L, _L)] for q in range(nq)]
                av1 = [colv[i + 1, pl.ds(q * _L, _L)] for q in range(nq)]

                def j_body(j, _, i=i, av0=av0, av1=av1):
                    for q in range(nq):
                        bv = rowv[j, pl.ds(q * _L, _L)]
                        outv[i, j, pl.ds(q * _L, _L)] = av0[q] + bv
                        outv[i + 1, j, pl.ds(q * _L, _L)] = av1[q] + bv
                    return 0

                lax.fori_loop(0, jpw, j_body, 0)
                return 0

            lax.fori_loop(0, half // 2, ig_body, 0)
            outs.append(pltpu.async_copy(
                outv.at[pl.ds(hi * half, half)],
                out_hbm.at[pl.ds(i0 + hi * half, half), pl.ds(j0, jpw)],
                sem_out))
        for c in outs:
            c.wait()

    return body(col_embed, row_embed)


def kernel(patch, row_embed, col_embed):
    hw = patch.shape[0]
    h = int(math.isqrt(hw))
    w = h
    d_model = row_embed.shape[1]
    pos = _pos_embed_sc(row_embed, col_embed, h, w)
    return jnp.transpose(pos, (2, 0, 1))[None]


# speedup vs baseline: 1.0046x; 1.0046x over previous
import functools
import math

import jax
import jax.numpy as jnp
from jax import lax
from jax.experimental import pallas as pl
from jax.experimental.pallas import tpu as pltpu
from jax.experimental.pallas import tpu_sc as plsc

_L = 16
_NC = 2
_NS = 16


@functools.partial(jax.jit, static_argnames=("h", "w"))
def _pos_embed_sc(row_embed, col_embed, h, w):
    d_model = row_embed.shape[1]
    ib_n, jb_n = 8, 4
    ipw = h // ib_n
    jpw = w // jb_n
    nq = d_model // _L

    mesh = plsc.VectorSubcoreMesh(
        core_axis_name="c", subcore_axis_name="s",
        num_cores=_NC, num_subcores=_NS,
    )

    @functools.partial(
        pl.kernel,
        out_type=jax.ShapeDtypeStruct((h, w, d_model), jnp.float32),
        mesh=mesh,
        scratch_types=[
            pltpu.VMEM((ipw, d_model), jnp.float32),
            pltpu.VMEM((jpw, d_model), jnp.float32),
            pltpu.VMEM((ipw, jpw, d_model), jnp.float32),
            pltpu.SemaphoreType.DMA,
            pltpu.SemaphoreType.DMA,
        ],
        compiler_params=pltpu.CompilerParams(
            needs_layout_passes=False,
            disable_bounds_checks=True,
            disable_semaphore_checks=True,
            skip_device_barrier=True,
        ),
    )
    def body(col_hbm, row_hbm, out_hbm, colv, rowv, outv, sem_in, sem_out):
        wid = lax.axis_index("s") * _NC + lax.axis_index("c")
        ib = wid // jb_n
        jb = wid - ib * jb_n
        i0 = ib * ipw
        j0 = jb * jpw
        cin0 = pltpu.async_copy(col_hbm.at[pl.ds(i0, ipw)], colv, sem_in)
        cin1 = pltpu.async_copy(row_hbm.at[pl.ds(j0, jpw)], rowv, sem_in)
        cin0.wait()
        cin1.wait()

        half = ipw // 2
        outs = []
        for hi in range(2):
            def ig_body(ig, _, hi=hi):
                i = hi * half + ig * 2
                av0 = [colv[i, pl.ds(q * _L, _L)] for q in range(nq)]
                av1 = [colv[i + 1, pl.ds(q * _L, _L)] for q in range(nq)]

                def j_body(j, _, i=i, av0=av0, av1=av1):
                    for q in range(nq):
                        bv = rowv[j, pl.ds(q * _L, _L)]
                        outv[i, j, pl.ds(q * _L, _L)] = av0[q] + bv
                        outv[i + 1, j, pl.ds(q * _L, _L)] = av1[q] + bv
                    return 0

                lax.fori_loop(0, jpw, j_body, 0)
                return 0

            lax.fori_loop(0, half // 2, ig_body, 0)
            outs.append(pltpu.async_copy(
                outv.at[pl.ds(hi * half, half)],
                out_hbm.at[pl.ds(i0 + hi * half, half), pl.ds(j0, jpw)],
                sem_out))
        for c in outs:
            c.wait()

    return body(col_embed, row_embed)


def kernel(patch, row_embed, col_embed):
    hw = patch.shape[0]
    h = int(math.isqrt(hw))
    w = h
    d_model = row_embed.shape[1]
    pos = _pos_embed_sc(row_embed, col_embed, h, w)
    return jnp.transpose(pos, (2, 0, 1))[None]


# speedup vs baseline: 1.1546x; 1.1494x over previous
import functools
import math

import jax
import jax.numpy as jnp
from jax import lax
from jax.experimental import pallas as pl
from jax.experimental.pallas import tpu as pltpu
from jax.experimental.pallas import tpu_sc as plsc

_L = 16
_NC = 2
_NS = 16


@functools.partial(jax.jit, static_argnames=("h", "w"))
def _pos_embed_sc(row_embed, col_embed, h, w):
    d_model = row_embed.shape[1]
    ib_n, jb_n = 8, 4
    ipw = h // ib_n
    jpw = w // jb_n
    nq = d_model // _L

    mesh = plsc.VectorSubcoreMesh(
        core_axis_name="c", subcore_axis_name="s",
        num_cores=_NC, num_subcores=_NS,
    )

    @functools.partial(
        pl.kernel,
        out_type=jax.ShapeDtypeStruct((h, w, d_model), jnp.float32),
        mesh=mesh,
        scratch_types=[
            pltpu.VMEM((ipw, d_model), jnp.float32),
            pltpu.VMEM((jpw, d_model), jnp.float32),
            pltpu.VMEM((ipw, jpw, d_model), jnp.float32),
            pltpu.SemaphoreType.DMA,
            pltpu.SemaphoreType.DMA,
        ],
        compiler_params=pltpu.CompilerParams(
            needs_layout_passes=False,
            disable_bounds_checks=True,
            disable_semaphore_checks=True,
            skip_device_barrier=True,
        ),
    )
    def body(col_hbm, row_hbm, out_hbm, colv, rowv, outv, sem_in, sem_out):
        wid = lax.axis_index("s") * _NC + lax.axis_index("c")
        ib = wid // jb_n
        jb = wid - ib * jb_n
        i0 = ib * ipw
        j0 = jb * jpw
        cin0 = pltpu.async_copy(col_hbm.at[pl.ds(i0, ipw)], colv, sem_in)
        cin1 = pltpu.async_copy(row_hbm.at[pl.ds(j0, jpw)], rowv, sem_in)
        cin0.wait()
        cin1.wait()

        half = ipw // 2
        outs = []
        for hi in range(2):
            @plsc.parallel_loop(hi * half, hi * half + half, 2)
            def _ig_body(i, hi=hi):
                av0 = [colv[i, pl.ds(q * _L, _L)] for q in range(nq)]
                av1 = [colv[i + 1, pl.ds(q * _L, _L)] for q in range(nq)]

                @plsc.parallel_loop(0, jpw, 1, unroll=2)
                def _j_body(j, i=i, av0=av0, av1=av1):
                    for q in range(nq):
                        bv = rowv[j, pl.ds(q * _L, _L)]
                        outv[i, j, pl.ds(q * _L, _L)] = av0[q] + bv
                        outv[i + 1, j, pl.ds(q * _L, _L)] = av1[q] + bv
            outs.append(pltpu.async_copy(
                outv.at[pl.ds(hi * half, half)],
                out_hbm.at[pl.ds(i0 + hi * half, half), pl.ds(j0, jpw)],
                sem_out))
        for c in outs:
            c.wait()

    return body(col_embed, row_embed)


def kernel(patch, row_embed, col_embed):
    hw = patch.shape[0]
    h = int(math.isqrt(hw))
    w = h
    d_model = row_embed.shape[1]
    pos = _pos_embed_sc(row_embed, col_embed, h, w)
    return jnp.transpose(pos, (2, 0, 1))[None]


# speedup vs baseline: 1.1575x; 1.0025x over previous
import functools
import math

import jax
import jax.numpy as jnp
from jax import lax
from jax.experimental import pallas as pl
from jax.experimental.pallas import tpu as pltpu
from jax.experimental.pallas import tpu_sc as plsc

_L = 16
_NC = 2
_NS = 16


@functools.partial(jax.jit, static_argnames=("h", "w"))
def _pos_embed_sc(row_embed, col_embed, h, w):
    d_model = row_embed.shape[1]
    ib_n, jb_n = 8, 4
    ipw = h // ib_n
    jpw = w // jb_n
    nq = d_model // _L

    mesh = plsc.VectorSubcoreMesh(
        core_axis_name="c", subcore_axis_name="s",
        num_cores=_NC, num_subcores=_NS,
    )

    @functools.partial(
        pl.kernel,
        out_type=jax.ShapeDtypeStruct((h, w, d_model), jnp.float32),
        mesh=mesh,
        scratch_types=[
            pltpu.VMEM((ipw, d_model), jnp.float32),
            pltpu.VMEM((jpw, d_model), jnp.float32),
            pltpu.VMEM((ipw, jpw, d_model), jnp.float32),
            pltpu.SemaphoreType.DMA,
            pltpu.SemaphoreType.DMA,
        ],
        compiler_params=pltpu.CompilerParams(
            needs_layout_passes=False,
            disable_bounds_checks=True,
            disable_semaphore_checks=True,
            skip_device_barrier=True,
        ),
    )
    def body(col_hbm, row_hbm, out_hbm, colv, rowv, outv, sem_in, sem_out):
        wid = lax.axis_index("s") * _NC + lax.axis_index("c")
        ib = wid // jb_n
        jb = wid - ib * jb_n
        i0 = ib * ipw
        j0 = jb * jpw
        cin0 = pltpu.async_copy(col_hbm.at[pl.ds(i0, ipw)], colv, sem_in)
        cin1 = pltpu.async_copy(row_hbm.at[pl.ds(j0, jpw)], rowv, sem_in)
        cin0.wait()
        cin1.wait()

        half = ipw // 4
        outs = []
        for hi in range(4):
            @plsc.parallel_loop(hi * half, hi * half + half, 2)
            def _ig_body(i, hi=hi):
                av0 = [colv[i, pl.ds(q * _L, _L)] for q in range(nq)]
                av1 = [colv[i + 1, pl.ds(q * _L, _L)] for q in range(nq)]

                @plsc.parallel_loop(0, jpw, 1, unroll=2)
                def _j_body(j, i=i, av0=av0, av1=av1):
                    for q in range(nq):
                        bv = rowv[j, pl.ds(q * _L, _L)]
                        outv[i, j, pl.ds(q * _L, _L)] = av0[q] + bv
                        outv[i + 1, j, pl.ds(q * _L, _L)] = av1[q] + bv
            outs.append(pltpu.async_copy(
                outv.at[pl.ds(hi * half, half)],
                out_hbm.at[pl.ds(i0 + hi * half, half), pl.ds(j0, jpw)],
                sem_out))
        for c in outs:
            c.wait()

    return body(col_embed, row_embed)


def kernel(patch, row_embed, col_embed):
    hw = patch.shape[0]
    h = int(math.isqrt(hw))
    w = h
    d_model = row_embed.shape[1]
    pos = _pos_embed_sc(row_embed, col_embed, h, w)
    return jnp.transpose(pos, (2, 0, 1))[None]
